# Initial kernel scaffold; baseline (speedup 1.0000x reference)
import jax
import jax.numpy as jnp
from jax.experimental import pallas as pl


def kernel(x, edge_index, batch, W1l, b1, W1r, W2l, b2, W2r, Wh, bh):
    raise NotImplementedError("write your pallas kernel here")


# speedup vs baseline: 1.8641x; 1.8641x over previous
import jax
import jax.numpy as jnp
from jax import lax
from jax.experimental import pallas as pl
from jax.experimental.pallas import tpu as pltpu
from jax.experimental.pallas import tpu_sc as plsc

N = 10000
D = 128
G = 64

NC = 2
NS = 16
NW = NC * NS
CH = 64
IB = 4

NA = N + 8
RPW = 632


def _make_sc_agg(nb):
    mesh = plsc.VectorSubcoreMesh(core_axis_name="c", subcore_axis_name="s",
                                  num_cores=NC, num_subcores=NS)
    scratch = [
        pltpu.VMEM((2 * IB, CH), jnp.int32),
        pltpu.VMEM((CH, D), jnp.float32),
        pltpu.VMEM_SHARED((NA, D), jnp.float32),
        pltpu.SemaphoreType.DMA,
    ]

    def body(table, eidx, z_d, agg_out, bufa, r0, acc, sem0):
        c = lax.axis_index("c")
        s = lax.axis_index("s")
        w = c * NS + s

        zlast = NA - (NS - 1) * RPW

        @pl.when(s < NS - 1)
        def _():
            pltpu.sync_copy(z_d, acc.at[pl.ds(s * RPW, RPW)])

        @pl.when(s == NS - 1)
        def _():
            pltpu.sync_copy(z_d.at[pl.ds(0, zlast)],
                            acc.at[pl.ds(s * RPW, zlast)])

        def sstep(b, carry):
            pltpu.sync_copy(eidx.at[w, b], bufa)
            for j in range(IB):
                pltpu.async_copy(table.at[bufa.at[j]], r0, sem0).wait()
                pltpu.sync_copy(r0, acc.at[bufa.at[IB + j]], add=True)
            return carry

        lax.fori_loop(0, nb, sstep, 0)
        plsc.subcore_barrier()

        olast = N - (NS - 1) * RPW

        @pl.when(s < NS - 1)
        def _():
            pltpu.sync_copy(acc.at[pl.ds(s * RPW, RPW)],
                            agg_out.at[c, pl.ds(s * RPW, RPW)])

        @pl.when(s == NS - 1)
        def _():
            pltpu.sync_copy(acc.at[pl.ds(s * RPW, olast)],
                            agg_out.at[c, pl.ds(s * RPW, olast)])

    return pl.kernel(body, out_type=jax.ShapeDtypeStruct((NC, N, D),
                                                         jnp.float32),
                     mesh=mesh, scratch_types=scratch)


def _tc_layer(agg_p, inv3, x, WlT, b, WrT):
    R = 1000

    def body(agg_ref, inv_ref, x_ref, wl_ref, b_ref, wr_ref, o_ref):
        agg = agg_ref[0] + agg_ref[1]
        mean = agg * inv_ref[0, 0, :][:, None]
        o = jnp.dot(mean, wl_ref[...], preferred_element_type=jnp.float32)
        o += b_ref[0][None, :]
        o += jnp.dot(x_ref[...], wr_ref[...], preferred_element_type=jnp.float32)
        o_ref[...] = jnp.maximum(o, 0.0)

    return pl.pallas_call(
        body,
        grid=(N // R,),
        in_specs=[
            pl.BlockSpec((NC, R, D), lambda i: (0, i, 0)),
            pl.BlockSpec((1, 1, R), lambda i: (i, 0, 0)),
            pl.BlockSpec((R, D), lambda i: (i, 0)),
            pl.BlockSpec((D, D), lambda i: (0, 0)),
            pl.BlockSpec((1, D), lambda i: (0, 0)),
            pl.BlockSpec((D, D), lambda i: (0, 0)),
        ],
        out_specs=pl.BlockSpec((R, D), lambda i: (i, 0)),
        out_shape=jax.ShapeDtypeStruct((N, D), jnp.float32),
    )(agg_p, inv3, x, WlT, b.reshape(1, D), WrT)


def _tc_final(agg_p, inv3, h, WlT, b, WrT, batch3, WhTp, bhp):
    R = 1000
    nblk = N // R

    def body(agg_ref, inv_ref, h_ref, wl_ref, b_ref, wr_ref, bt_ref,
             wh_ref, bh_ref, o_ref, pool_ref, cnt_ref):
        i = pl.program_id(0)
        agg = agg_ref[0] + agg_ref[1]
        mean = agg * inv_ref[0, 0, :][:, None]
        h2 = jnp.dot(mean, wl_ref[...], preferred_element_type=jnp.float32)
        h2 += b_ref[0][None, :]
        h2 += jnp.dot(h_ref[...], wr_ref[...], preferred_element_type=jnp.float32)
        h2 = jnp.maximum(h2, 0.0)
        bt = bt_ref[0, 0, :]
        onehot = (bt[None, :] == lax.broadcasted_iota(jnp.int32, (G, R), 0)
                  ).astype(jnp.float32)
        contrib = jnp.dot(onehot, h2, preferred_element_type=jnp.float32)
        cnt = jnp.sum(onehot, axis=1)[:, None]

        @pl.when(i == 0)
        def _():
            pool_ref[...] = jnp.zeros_like(pool_ref)
            cnt_ref[...] = jnp.zeros_like(cnt_ref)

        pool_ref[...] += contrib
        cnt_ref[...] += jnp.broadcast_to(cnt, (G, D))

        @pl.when(i == nblk - 1)
        def _():
            pooled = pool_ref[...] / jnp.maximum(cnt_ref[...], 1.0)
            o_ref[...] = (jnp.dot(pooled, wh_ref[...],
                                  preferred_element_type=jnp.float32)
                          + bh_ref[0][None, :])

    return pl.pallas_call(
        body,
        grid=(nblk,),
        in_specs=[
            pl.BlockSpec((NC, R, D), lambda i: (0, i, 0)),
            pl.BlockSpec((1, 1, R), lambda i: (i, 0, 0)),
            pl.BlockSpec((R, D), lambda i: (i, 0)),
            pl.BlockSpec((D, D), lambda i: (0, 0)),
            pl.BlockSpec((1, D), lambda i: (0, 0)),
            pl.BlockSpec((D, D), lambda i: (0, 0)),
            pl.BlockSpec((1, 1, R), lambda i: (i, 0, 0)),
            pl.BlockSpec((D, D), lambda i: (0, 0)),
            pl.BlockSpec((1, D), lambda i: (0, 0)),
        ],
        out_specs=pl.BlockSpec((G, D), lambda i: (0, 0)),
        out_shape=jax.ShapeDtypeStruct((G, D), jnp.float32),
        scratch_shapes=[
            pltpu.VMEM((G, D), jnp.float32),
            pltpu.VMEM((G, D), jnp.float32),
        ],
    )(agg_p, inv3, h, WlT, b.reshape(1, D), WrT, batch3, WhTp, bhp)


def kernel(x, edge_index, batch, W1l, b1, W1r, W2l, b2, W2r, Wh, bh):
    E = edge_index.shape[1]
    epb = IB * CH
    nb = -(-E // (NW * epb))
    e_pad = NW * nb * epb

    src = edge_index[0]
    dst = edge_index[1]
    order = jnp.argsort(dst)
    dst_s = dst[order]
    pad = e_pad - E
    src_p = jnp.concatenate([src[order], jnp.zeros((pad,), jnp.int32)])
    dst_p = jnp.concatenate([dst_s, jnp.full((pad,), N, jnp.int32)])
    src4 = src_p.reshape(NW, nb, IB, CH)
    dst4 = dst_p.reshape(NW, nb, IB, CH)
    eidx = jnp.concatenate([src4, dst4], axis=2)

    ids = jnp.arange(N, dtype=jnp.int32)
    deg = (jnp.searchsorted(dst_s, ids, side="right")
           - jnp.searchsorted(dst_s, ids, side="left")).astype(jnp.float32)
    inv3 = (1.0 / jnp.maximum(deg, 1.0)).reshape(N // 1000, 1, 1000)

    z_d = jnp.zeros((RPW, D), jnp.float32)

    agg1_p = _make_sc_agg(nb)(x, eidx, z_d)
    h = _tc_layer(agg1_p, inv3, x, W1l.T, b1, W1r.T)
    agg2_p = _make_sc_agg(nb)(h, eidx, z_d)

    batch3 = batch.reshape(N // 1000, 1, 1000)
    WhTp = jnp.zeros((D, D), jnp.float32).at[:3, :].set(Wh).T
    bhp = jnp.zeros((1, D), jnp.float32).at[0, :3].set(bh)
    outp = _tc_final(agg2_p, inv3, h, W2l.T, b2, W2r.T, batch3, WhTp, bhp)
    return outp[:, :3]


# speedup vs baseline: 1.9687x; 1.0561x over previous
import jax
import jax.numpy as jnp
from jax import lax
from jax.experimental import pallas as pl
from jax.experimental.pallas import tpu as pltpu
from jax.experimental.pallas import tpu_sc as plsc

N = 10000
D = 128
G = 64

NC = 2
NS = 16
NW = NC * NS
CH = 64
IB = 4

NA = N + 8
RPW = 632


def _make_sc_agg(nb):
    mesh = plsc.VectorSubcoreMesh(core_axis_name="c", subcore_axis_name="s",
                                  num_cores=NC, num_subcores=NS)
    scratch = [
        pltpu.VMEM((2 * IB, CH), jnp.int32),
        pltpu.VMEM((2 * IB, CH), jnp.int32),
        [pltpu.VMEM((CH, D), jnp.float32) for _ in range(IB)],
        pltpu.VMEM_SHARED((NA, D), jnp.float32),
        [pltpu.SemaphoreType.DMA for _ in range(IB)],
        pltpu.SemaphoreType.DMA,
    ]

    def body(table, eidx, z_d, agg_out, bufa, bufb, rbufs, acc, sems, semi):
        c = lax.axis_index("c")
        s = lax.axis_index("s")
        w = c * NS + s

        zlast = NA - (NS - 1) * RPW

        @pl.when(s < NS - 1)
        def _():
            pltpu.sync_copy(z_d, acc.at[pl.ds(s * RPW, RPW)])

        @pl.when(s == NS - 1)
        def _():
            pltpu.sync_copy(z_d.at[pl.ds(0, zlast)],
                            acc.at[pl.ds(s * RPW, zlast)])

        def proc(buf, nxt, pb):
            pltpu.async_copy(eidx.at[w, pb], nxt, semi)
            for j in range(IB):
                pltpu.async_copy(table.at[buf.at[j]], rbufs[j], sems[j])
            pltpu.make_async_copy(eidx.at[w, pb], nxt, semi).wait()
            for j in range(IB):
                pltpu.make_async_copy(table.at[buf.at[j]], rbufs[j],
                                      sems[j]).wait()
            for j in range(IB):
                pltpu.sync_copy(rbufs[j], acc.at[buf.at[IB + j]], add=True)

        pltpu.sync_copy(eidx.at[w, 0], bufa)

        def step(k, carry):
            b = 2 * k
            proc(bufa, bufb, b + 1)
            proc(bufb, bufa, jnp.minimum(b + 2, nb - 1))
            return carry

        lax.fori_loop(0, nb // 2, step, 0)
        plsc.subcore_barrier()

        olast = N - (NS - 1) * RPW

        @pl.when(s < NS - 1)
        def _():
            pltpu.sync_copy(acc.at[pl.ds(s * RPW, RPW)],
                            agg_out.at[c, pl.ds(s * RPW, RPW)])

        @pl.when(s == NS - 1)
        def _():
            pltpu.sync_copy(acc.at[pl.ds(s * RPW, olast)],
                            agg_out.at[c, pl.ds(s * RPW, olast)])

    return pl.kernel(body, out_type=jax.ShapeDtypeStruct((NC, N, D),
                                                         jnp.float32),
                     mesh=mesh, scratch_types=scratch)


def _tc_layer(agg_p, inv3, x, WlT, b, WrT):
    R = 1000

    def body(agg_ref, inv_ref, x_ref, wl_ref, b_ref, wr_ref, o_ref):
        agg = agg_ref[0] + agg_ref[1]
        mean = agg * inv_ref[0, 0, :][:, None]
        o = jnp.dot(mean, wl_ref[...], preferred_element_type=jnp.float32)
        o += b_ref[0][None, :]
        o += jnp.dot(x_ref[...], wr_ref[...], preferred_element_type=jnp.float32)
        o_ref[...] = jnp.maximum(o, 0.0)

    return pl.pallas_call(
        body,
        grid=(N // R,),
        in_specs=[
            pl.BlockSpec((NC, R, D), lambda i: (0, i, 0)),
            pl.BlockSpec((1, 1, R), lambda i: (i, 0, 0)),
            pl.BlockSpec((R, D), lambda i: (i, 0)),
            pl.BlockSpec((D, D), lambda i: (0, 0)),
            pl.BlockSpec((1, D), lambda i: (0, 0)),
            pl.BlockSpec((D, D), lambda i: (0, 0)),
        ],
        out_specs=pl.BlockSpec((R, D), lambda i: (i, 0)),
        out_shape=jax.ShapeDtypeStruct((N, D), jnp.float32),
    )(agg_p, inv3, x, WlT, b.reshape(1, D), WrT)


def _tc_final(agg_p, inv3, h, WlT, b, WrT, batch3, WhTp, bhp):
    R = 1000
    nblk = N // R

    def body(agg_ref, inv_ref, h_ref, wl_ref, b_ref, wr_ref, bt_ref,
             wh_ref, bh_ref, o_ref, pool_ref, cnt_ref):
        i = pl.program_id(0)
        agg = agg_ref[0] + agg_ref[1]
        mean = agg * inv_ref[0, 0, :][:, None]
        h2 = jnp.dot(mean, wl_ref[...], preferred_element_type=jnp.float32)
        h2 += b_ref[0][None, :]
        h2 += jnp.dot(h_ref[...], wr_ref[...], preferred_element_type=jnp.float32)
        h2 = jnp.maximum(h2, 0.0)
        bt = bt_ref[0, 0, :]
        onehot = (bt[None, :] == lax.broadcasted_iota(jnp.int32, (G, R), 0)
                  ).astype(jnp.float32)
        contrib = jnp.dot(onehot, h2, preferred_element_type=jnp.float32)
        cnt = jnp.sum(onehot, axis=1)[:, None]

        @pl.when(i == 0)
        def _():
            pool_ref[...] = jnp.zeros_like(pool_ref)
            cnt_ref[...] = jnp.zeros_like(cnt_ref)

        pool_ref[...] += contrib
        cnt_ref[...] += jnp.broadcast_to(cnt, (G, D))

        @pl.when(i == nblk - 1)
        def _():
            pooled = pool_ref[...] / jnp.maximum(cnt_ref[...], 1.0)
            o_ref[...] = (jnp.dot(pooled, wh_ref[...],
                                  preferred_element_type=jnp.float32)
                          + bh_ref[0][None, :])

    return pl.pallas_call(
        body,
        grid=(nblk,),
        in_specs=[
            pl.BlockSpec((NC, R, D), lambda i: (0, i, 0)),
            pl.BlockSpec((1, 1, R), lambda i: (i, 0, 0)),
            pl.BlockSpec((R, D), lambda i: (i, 0)),
            pl.BlockSpec((D, D), lambda i: (0, 0)),
            pl.BlockSpec((1, D), lambda i: (0, 0)),
            pl.BlockSpec((D, D), lambda i: (0, 0)),
            pl.BlockSpec((1, 1, R), lambda i: (i, 0, 0)),
            pl.BlockSpec((D, D), lambda i: (0, 0)),
            pl.BlockSpec((1, D), lambda i: (0, 0)),
        ],
        out_specs=pl.BlockSpec((G, D), lambda i: (0, 0)),
        out_shape=jax.ShapeDtypeStruct((G, D), jnp.float32),
        scratch_shapes=[
            pltpu.VMEM((G, D), jnp.float32),
            pltpu.VMEM((G, D), jnp.float32),
        ],
    )(agg_p, inv3, h, WlT, b.reshape(1, D), WrT, batch3, WhTp, bhp)


def kernel(x, edge_index, batch, W1l, b1, W1r, W2l, b2, W2r, Wh, bh):
    E = edge_index.shape[1]
    epb = IB * CH
    nb = -(-E // (NW * epb))
    if nb % 2:
        nb += 1
    e_pad = NW * nb * epb

    src = edge_index[0]
    dst = edge_index[1]
    order = jnp.argsort(dst)
    dst_s = dst[order]
    pad = e_pad - E
    src_p = jnp.concatenate([src[order], jnp.zeros((pad,), jnp.int32)])
    dst_p = jnp.concatenate([dst_s, jnp.full((pad,), N, jnp.int32)])
    src4 = src_p.reshape(NW, nb, IB, CH)
    dst4 = dst_p.reshape(NW, nb, IB, CH)
    eidx = jnp.concatenate([src4, dst4], axis=2)

    ids = jnp.arange(N, dtype=jnp.int32)
    deg = (jnp.searchsorted(dst_s, ids, side="right")
           - jnp.searchsorted(dst_s, ids, side="left")).astype(jnp.float32)
    inv3 = (1.0 / jnp.maximum(deg, 1.0)).reshape(N // 1000, 1, 1000)

    z_d = jnp.zeros((RPW, D), jnp.float32)

    agg1_p = _make_sc_agg(nb)(x, eidx, z_d)
    h = _tc_layer(agg1_p, inv3, x, W1l.T, b1, W1r.T)
    agg2_p = _make_sc_agg(nb)(h, eidx, z_d)

    batch3 = batch.reshape(N // 1000, 1, 1000)
    WhTp = jnp.zeros((D, D), jnp.float32).at[:3, :].set(Wh).T
    bhp = jnp.zeros((1, D), jnp.float32).at[0, :3].set(bh)
    outp = _tc_final(agg2_p, inv3, h, W2l.T, b2, W2r.T, batch3, WhTp, bhp)
    return outp[:, :3]


# speedup vs baseline: 2.0122x; 1.0221x over previous
import jax
import jax.numpy as jnp
from jax import lax
from jax.experimental import pallas as pl
from jax.experimental.pallas import tpu as pltpu
from jax.experimental.pallas import tpu_sc as plsc

N = 10000
D = 128
G = 64

NC = 2
NS = 16
NW = NC * NS
CH = 128
IB = 2

NA = N + 8
RPW = 632


def _make_sc_agg(nb):
    mesh = plsc.VectorSubcoreMesh(core_axis_name="c", subcore_axis_name="s",
                                  num_cores=NC, num_subcores=NS)
    scratch = [
        pltpu.VMEM((2 * IB, CH), jnp.int32),
        pltpu.VMEM((2 * IB, CH), jnp.int32),
        [pltpu.VMEM((CH, D), jnp.float32) for _ in range(IB)],
        pltpu.VMEM_SHARED((NA, D), jnp.float32),
        [pltpu.SemaphoreType.DMA for _ in range(IB)],
        pltpu.SemaphoreType.DMA,
    ]

    def body(table, eidx, z_d, agg_out, bufa, bufb, rbufs, acc, sems, semi):
        c = lax.axis_index("c")
        s = lax.axis_index("s")
        w = c * NS + s

        zlast = NA - (NS - 1) * RPW

        @pl.when(s < NS - 1)
        def _():
            pltpu.sync_copy(z_d, acc.at[pl.ds(s * RPW, RPW)])

        @pl.when(s == NS - 1)
        def _():
            pltpu.sync_copy(z_d.at[pl.ds(0, zlast)],
                            acc.at[pl.ds(s * RPW, zlast)])

        def proc(buf, nxt, pb):
            pltpu.async_copy(eidx.at[w, pb], nxt, semi)
            for j in range(IB):
                pltpu.async_copy(table.at[buf.at[j]], rbufs[j], sems[j])
            pltpu.make_async_copy(eidx.at[w, pb], nxt, semi).wait()
            for j in range(IB):
                pltpu.make_async_copy(table.at[buf.at[j]], rbufs[j],
                                      sems[j]).wait()
            for j in range(IB):
                pltpu.sync_copy(rbufs[j], acc.at[buf.at[IB + j]], add=True)

        pltpu.sync_copy(eidx.at[w, 0], bufa)

        def step(k, carry):
            b = 2 * k
            proc(bufa, bufb, b + 1)
            proc(bufb, bufa, jnp.minimum(b + 2, nb - 1))
            return carry

        lax.fori_loop(0, nb // 2, step, 0)
        plsc.subcore_barrier()

        olast = N - (NS - 1) * RPW

        @pl.when(s < NS - 1)
        def _():
            pltpu.sync_copy(acc.at[pl.ds(s * RPW, RPW)],
                            agg_out.at[c, pl.ds(s * RPW, RPW)])

        @pl.when(s == NS - 1)
        def _():
            pltpu.sync_copy(acc.at[pl.ds(s * RPW, olast)],
                            agg_out.at[c, pl.ds(s * RPW, olast)])

    return pl.kernel(body, out_type=jax.ShapeDtypeStruct((NC, N, D),
                                                         jnp.float32),
                     mesh=mesh, scratch_types=scratch)


def _tc_layer(agg_p, inv3, x, WlT, b, WrT):
    R = 1000

    def body(agg_ref, inv_ref, x_ref, wl_ref, b_ref, wr_ref, o_ref):
        agg = agg_ref[0] + agg_ref[1]
        mean = agg * inv_ref[0, 0, :][:, None]
        o = jnp.dot(mean, wl_ref[...], preferred_element_type=jnp.float32)
        o += b_ref[0][None, :]
        o += jnp.dot(x_ref[...], wr_ref[...], preferred_element_type=jnp.float32)
        o_ref[...] = jnp.maximum(o, 0.0)

    return pl.pallas_call(
        body,
        grid=(N // R,),
        in_specs=[
            pl.BlockSpec((NC, R, D), lambda i: (0, i, 0)),
            pl.BlockSpec((1, 1, R), lambda i: (i, 0, 0)),
            pl.BlockSpec((R, D), lambda i: (i, 0)),
            pl.BlockSpec((D, D), lambda i: (0, 0)),
            pl.BlockSpec((1, D), lambda i: (0, 0)),
            pl.BlockSpec((D, D), lambda i: (0, 0)),
        ],
        out_specs=pl.BlockSpec((R, D), lambda i: (i, 0)),
        out_shape=jax.ShapeDtypeStruct((N, D), jnp.float32),
    )(agg_p, inv3, x, WlT, b.reshape(1, D), WrT)


def _tc_final(agg_p, inv3, h, WlT, b, WrT, batch3, WhTp, bhp):
    R = 1000
    nblk = N // R

    def body(agg_ref, inv_ref, h_ref, wl_ref, b_ref, wr_ref, bt_ref,
             wh_ref, bh_ref, o_ref, pool_ref, cnt_ref):
        i = pl.program_id(0)
        agg = agg_ref[0] + agg_ref[1]
        mean = agg * inv_ref[0, 0, :][:, None]
        h2 = jnp.dot(mean, wl_ref[...], preferred_element_type=jnp.float32)
        h2 += b_ref[0][None, :]
        h2 += jnp.dot(h_ref[...], wr_ref[...], preferred_element_type=jnp.float32)
        h2 = jnp.maximum(h2, 0.0)
        bt = bt_ref[0, 0, :]
        onehot = (bt[None, :] == lax.broadcasted_iota(jnp.int32, (G, R), 0)
                  ).astype(jnp.float32)
        contrib = jnp.dot(onehot, h2, preferred_element_type=jnp.float32)
        cnt = jnp.sum(onehot, axis=1)[:, None]

        @pl.when(i == 0)
        def _():
            pool_ref[...] = jnp.zeros_like(pool_ref)
            cnt_ref[...] = jnp.zeros_like(cnt_ref)

        pool_ref[...] += contrib
        cnt_ref[...] += jnp.broadcast_to(cnt, (G, D))

        @pl.when(i == nblk - 1)
        def _():
            pooled = pool_ref[...] / jnp.maximum(cnt_ref[...], 1.0)
            o_ref[...] = (jnp.dot(pooled, wh_ref[...],
                                  preferred_element_type=jnp.float32)
                          + bh_ref[0][None, :])

    return pl.pallas_call(
        body,
        grid=(nblk,),
        in_specs=[
            pl.BlockSpec((NC, R, D), lambda i: (0, i, 0)),
            pl.BlockSpec((1, 1, R), lambda i: (i, 0, 0)),
            pl.BlockSpec((R, D), lambda i: (i, 0)),
            pl.BlockSpec((D, D), lambda i: (0, 0)),
            pl.BlockSpec((1, D), lambda i: (0, 0)),
            pl.BlockSpec((D, D), lambda i: (0, 0)),
            pl.BlockSpec((1, 1, R), lambda i: (i, 0, 0)),
            pl.BlockSpec((D, D), lambda i: (0, 0)),
            pl.BlockSpec((1, D), lambda i: (0, 0)),
        ],
        out_specs=pl.BlockSpec((G, D), lambda i: (0, 0)),
        out_shape=jax.ShapeDtypeStruct((G, D), jnp.float32),
        scratch_shapes=[
            pltpu.VMEM((G, D), jnp.float32),
            pltpu.VMEM((G, D), jnp.float32),
        ],
    )(agg_p, inv3, h, WlT, b.reshape(1, D), WrT, batch3, WhTp, bhp)


def kernel(x, edge_index, batch, W1l, b1, W1r, W2l, b2, W2r, Wh, bh):
    E = edge_index.shape[1]
    epb = IB * CH
    nb = -(-E // (NW * epb))
    if nb % 2:
        nb += 1
    e_pad = NW * nb * epb

    src = edge_index[0]
    dst = edge_index[1]
    order = jnp.argsort(dst)
    dst_s = dst[order]
    pad = e_pad - E
    src_p = jnp.concatenate([src[order], jnp.zeros((pad,), jnp.int32)])
    dst_p = jnp.concatenate([dst_s, jnp.full((pad,), N, jnp.int32)])
    src4 = src_p.reshape(NW, nb, IB, CH)
    dst4 = dst_p.reshape(NW, nb, IB, CH)
    eidx = jnp.concatenate([src4, dst4], axis=2)

    ids = jnp.arange(N, dtype=jnp.int32)
    deg = (jnp.searchsorted(dst_s, ids, side="right")
           - jnp.searchsorted(dst_s, ids, side="left")).astype(jnp.float32)
    inv3 = (1.0 / jnp.maximum(deg, 1.0)).reshape(N // 1000, 1, 1000)

    z_d = jnp.zeros((RPW, D), jnp.float32)

    agg1_p = _make_sc_agg(nb)(x, eidx, z_d)
    h = _tc_layer(agg1_p, inv3, x, W1l.T, b1, W1r.T)
    agg2_p = _make_sc_agg(nb)(h, eidx, z_d)

    batch3 = batch.reshape(N // 1000, 1, 1000)
    WhTp = jnp.zeros((D, D), jnp.float32).at[:3, :].set(Wh).T
    bhp = jnp.zeros((1, D), jnp.float32).at[0, :3].set(bh)
    outp = _tc_final(agg2_p, inv3, h, W2l.T, b2, W2r.T, batch3, WhTp, bhp)
    return outp[:, :3]
